# baseline (speedup 1.0000x reference)
import functools

import jax
import jax.numpy as jnp
from jax import lax
from jax.experimental import pallas as pl
from jax.experimental.pallas import tpu as pltpu
from jax.experimental.pallas import tpu_sc as plsc

R = 16384
C = 4096
K = 3
NW = 32
EPW = R * K // NW
CHUNK = 128
NCHUNK = EPW // CHUNK
COLS = (1, 3, 0)


@functools.partial(
    pl.kernel,
    out_type=jax.ShapeDtypeStruct((R * K,), jnp.float32),
    mesh=plsc.VectorSubcoreMesh(core_axis_name="c", subcore_axis_name="s"),
    scratch_types=[
        pltpu.VMEM((EPW,), jnp.int32),
        pltpu.VMEM((EPW,), jnp.float32),
        pltpu.SemaphoreType.DMA,
    ],
)
def _sc_gather(x_hbm, idx_hbm, out_hbm, idx_v, out_v, sem):
    wid = lax.axis_index("s") * 2 + lax.axis_index("c")
    base = wid * EPW

    pltpu.sync_copy(idx_hbm.at[pl.ds(base, EPW)], idx_v)

    copies = []
    for ch in range(NCHUNK):
        copies.append(
            pltpu.async_copy(
                x_hbm.at[idx_v.at[pl.ds(ch * CHUNK, CHUNK)]],
                out_v.at[pl.ds(ch * CHUNK, CHUNK)],
                sem,
            )
        )
    for cp in copies:
        cp.wait()

    pltpu.sync_copy(out_v, out_hbm.at[pl.ds(base, EPW)])


def kernel(x):
    x_flat = x.reshape(R * C)
    rows = jnp.repeat(jnp.arange(R, dtype=jnp.int32), K)
    cols = jnp.tile(jnp.array(COLS, dtype=jnp.int32), R)
    idx = rows * C + cols
    out_flat = _sc_gather(x_flat, idx)
    return out_flat.reshape(R, K)


# speedup vs baseline: 8.8093x; 8.8093x over previous
import functools

import jax
import jax.numpy as jnp
from jax import lax
from jax.experimental import pallas as pl
from jax.experimental.pallas import tpu as pltpu
from jax.experimental.pallas import tpu_sc as plsc

R = 16384
C = 4096
K = 3
NW = 32
RPW = R // NW
BPW = RPW // 8
L = 16
COLS = (1, 3, 0)


@functools.partial(
    pl.kernel,
    out_type=jax.ShapeDtypeStruct((K * R,), jnp.float32),
    mesh=plsc.VectorSubcoreMesh(core_axis_name="c", subcore_axis_name="s"),
    scratch_types=[
        pltpu.VMEM((BPW, 8, 128), jnp.float32),
        pltpu.VMEM((RPW,), jnp.float32),
        pltpu.VMEM((RPW,), jnp.float32),
        pltpu.VMEM((RPW,), jnp.float32),
    ],
    compiler_params=pltpu.CompilerParams(
        use_tc_tiling_on_sc=True,
        needs_layout_passes=False,
    ),
)
def _sc_gather(x_hbm, out_hbm, blk_v, col0_v, col1_v, col2_v, sem=None):
    wid = lax.axis_index("s") * 2 + lax.axis_index("c")
    base = wid * RPW

    pltpu.sync_copy(
        x_hbm.at[pl.ds(wid * BPW, BPW), :, pl.ds(0, 128)], blk_v
    )

    iota = lax.iota(jnp.int32, L)
    bufs = (col0_v, col1_v, col2_v)

    def extract(k, carry):
        m = k * L + iota
        b = lax.shift_right_logical(m, 3)
        r8 = lax.bitwise_and(m, 7)
        for jj, j in enumerate(COLS):
            lane = jnp.full((L,), j, jnp.int32)
            vals = plsc.load_gather(blk_v, [b, r8, lane])
            bufs[jj][pl.ds(k * L, L)] = vals
        return carry

    lax.fori_loop(0, RPW // L, extract, None)

    for jj in range(K):
        pltpu.sync_copy(bufs[jj], out_hbm.at[pl.ds(jj * R + base, RPW)])


def kernel(x):
    x3 = x.reshape(R // 8, 8, C)
    out_flat = _sc_gather(x3)
    return out_flat.reshape(K, R).T


# speedup vs baseline: 9.5237x; 1.0811x over previous
import functools

import jax
import jax.numpy as jnp
from jax import lax
from jax.experimental import pallas as pl
from jax.experimental.pallas import tpu as pltpu
from jax.experimental.pallas import tpu_sc as plsc

R = 16384
C = 4096
K = 3
KP = 4
NW = 32
RPW = R // NW
BPW = RPW // 8
L = 16
COLS = (1, 3, 0)
OPW = RPW // 128 * KP * 128


@functools.partial(
    pl.kernel,
    out_type=jax.ShapeDtypeStruct((KP * R,), jnp.float32),
    mesh=plsc.VectorSubcoreMesh(core_axis_name="c", subcore_axis_name="s"),
    scratch_types=[
        pltpu.VMEM((BPW, 8, 128), jnp.float32),
        pltpu.VMEM((OPW,), jnp.float32),
    ],
    compiler_params=pltpu.CompilerParams(
        use_tc_tiling_on_sc=True,
        needs_layout_passes=False,
    ),
)
def _sc_gather(x_hbm, out_hbm, blk_v, out_v):
    wid = lax.axis_index("s") * 2 + lax.axis_index("c")

    pltpu.sync_copy(
        x_hbm.at[pl.ds(wid * BPW, BPW), :, pl.ds(0, 128)], blk_v
    )

    iota = lax.iota(jnp.int32, L)

    def extract(k, carry):
        m = k * L + iota
        b = lax.shift_right_logical(m, 3)
        r8 = lax.bitwise_and(m, 7)
        for jj, j in enumerate(COLS):
            lane = jnp.full((L,), j, jnp.int32)
            vals = plsc.load_gather(blk_v, [b, r8, lane])
            off = (k // 8) * (KP * 128) + jj * 128 + (k % 8) * L
            out_v[pl.ds(off, L)] = vals
        return carry

    lax.fori_loop(0, RPW // L, extract, None)

    pltpu.sync_copy(out_v, out_hbm.at[pl.ds(wid * OPW, OPW)])


def kernel(x):
    x3 = x.reshape(R // 8, 8, C)
    out_flat = _sc_gather(x3)
    out = out_flat.reshape(R // 128, KP, 128).transpose(0, 2, 1)
    return out.reshape(R, KP)[:, :K]


# speedup vs baseline: 10.5214x; 1.1048x over previous
import functools

import jax
import jax.numpy as jnp
from jax import lax
from jax.experimental import pallas as pl
from jax.experimental.pallas import tpu as pltpu
from jax.experimental.pallas import tpu_sc as plsc

R = 16384
C = 4096
K = 3
KP = 4
NW = 32
RPW = R // NW
BPW = RPW // 8
L = 16
COLS = (1, 3, 0)
OPW = RPW // 128 * KP * 128


@functools.partial(
    pl.kernel,
    out_type=jax.ShapeDtypeStruct((KP * R,), jnp.float32),
    mesh=plsc.VectorSubcoreMesh(core_axis_name="c", subcore_axis_name="s"),
    scratch_types=[
        pltpu.VMEM((BPW, 8, 128), jnp.float32),
        pltpu.VMEM((OPW,), jnp.float32),
    ],
    compiler_params=pltpu.CompilerParams(
        use_tc_tiling_on_sc=True,
        needs_layout_passes=False,
    ),
)
def _sc_gather(x_hbm, out_hbm, blk_v, out_v):
    wid = lax.axis_index("s") * 2 + lax.axis_index("c")

    pltpu.sync_copy(
        x_hbm.at[pl.ds(wid * BPW, BPW), :, pl.ds(0, 16)],
        blk_v.at[:, :, pl.ds(0, 16)],
    )

    iota = lax.iota(jnp.int32, L)

    def extract(k, carry):
        m = k * L + iota
        b = lax.shift_right_logical(m, 3)
        r8 = lax.bitwise_and(m, 7)
        for jj, j in enumerate(COLS):
            lane = jnp.full((L,), j, jnp.int32)
            vals = plsc.load_gather(blk_v, [b, r8, lane])
            off = (k // 8) * (KP * 128) + jj * 128 + (k % 8) * L
            out_v[pl.ds(off, L)] = vals
        return carry

    lax.fori_loop(0, RPW // L, extract, None)

    pltpu.sync_copy(out_v, out_hbm.at[pl.ds(wid * OPW, OPW)])


def kernel(x):
    x3 = x.reshape(R // 8, 8, C)
    out_flat = _sc_gather(x3)
    out = out_flat.reshape(R // 128, KP, 128).transpose(0, 2, 1)
    return out.reshape(R, KP)[:, :K]
